# baseline (speedup 1.0000x reference)
import functools

import jax
import jax.numpy as jnp
from jax import lax
from jax.experimental import pallas as pl
from jax.experimental.pallas import tpu as pltpu
from jax.experimental.pallas import tpu_sc as plsc

B = 2
N = 2048
D = 128
NCORE = 2
NSUB = 16
NWORK = NCORE * NSUB
ROWS_PER_SUB = N // NSUB
CAP = 4096
CAPB = CAP + ROWS_PER_SUB * 16 + 16
ESLOT = NWORK * CAP
RAD2_T = 0.0024999999441206455

_MESH = plsc.VectorSubcoreMesh(
    core_axis_name="c", subcore_axis_name="s", num_cores=NCORE, num_subcores=NSUB
)


def _compact_body(px_hbm, py_hbm, src_hbm, dst_hbm, dx_hbm, dy_hbm, cnt_hbm,
                  pxv, pyv, sbuf, dbuf, xbuf, ybuf, cbuf):
    c = lax.axis_index("c")
    s = lax.axis_index("s")
    r = c * NSUB + s

    pltpu.sync_copy(px_hbm.at[pl.ds(c * N, N)], pxv)
    pltpu.sync_copy(py_hbm.at[pl.ds(c * N, N)], pyv)

    zf = jnp.zeros((16,), jnp.float32)
    zi = jnp.zeros((16,), jnp.int32)
    dumm = jnp.full((16,), N, jnp.int32)

    def prefill(k, _):
        o = k * 16
        sbuf[pl.ds(o, 16)] = zi
        dbuf[pl.ds(o, 16)] = dumm
        xbuf[pl.ds(o, 16)] = zf
        ybuf[pl.ds(o, 16)] = zf
        return 0

    lax.fori_loop(0, CAP // 16, prefill, 0)

    iota = lax.iota(jnp.int32, 16)

    def row_body(i, cnt):
        row = s * ROWS_PER_SUB + i
        rowv = jnp.full((16,), row, jnp.int32)
        base = (row // 16) * 16
        lane = jnp.full((16,), row % 16, jnp.int32)
        pix = jnp.take_along_axis(pxv[pl.ds(base, 16)], lane, axis=0)
        piy = jnp.take_along_axis(pyv[pl.ds(base, 16)], lane, axis=0)

        def chunk_body(ch, cnt):
            j0 = ch * 16
            pjx = pxv[pl.ds(j0, 16)]
            pjy = pyv[pl.ds(j0, 16)]
            dx = pjx - pix
            dy = pjy - piy
            m = dx * dx + dy * dy + jnp.float32(1e-8)
            msk = m < jnp.float32(RAD2_T)
            plsc.store_compressed(sbuf.at[pl.ds(cnt, 16)],
                                  iota + (j0 + c * N), mask=msk)
            plsc.store_compressed(dbuf.at[pl.ds(cnt, 16)], rowv, mask=msk)
            plsc.store_compressed(xbuf.at[pl.ds(cnt, 16)], dx, mask=msk)
            plsc.store_compressed(ybuf.at[pl.ds(cnt, 16)], dy, mask=msk)
            return cnt + jnp.sum(msk.astype(jnp.int32))

        cnt = lax.fori_loop(0, N // 16, chunk_body, cnt)
        return jnp.minimum(cnt, CAP)

    cnt = lax.fori_loop(0, ROWS_PER_SUB, row_body, jnp.int32(0))

    sbuf[pl.ds(cnt, 16)] = zi
    dbuf[pl.ds(cnt, 16)] = dumm
    xbuf[pl.ds(cnt, 16)] = zf
    ybuf[pl.ds(cnt, 16)] = zf

    o = r * CAP
    pltpu.sync_copy(sbuf.at[pl.ds(0, CAP)], src_hbm.at[pl.ds(o, CAP)])
    pltpu.sync_copy(dbuf.at[pl.ds(0, CAP)], dst_hbm.at[pl.ds(o, CAP)])
    pltpu.sync_copy(xbuf.at[pl.ds(0, CAP)], dx_hbm.at[pl.ds(o, CAP)])
    pltpu.sync_copy(ybuf.at[pl.ds(0, CAP)], dy_hbm.at[pl.ds(o, CAP)])
    cbuf[pl.ds(0, 16)] = jnp.full((16,), cnt, jnp.int32)
    pltpu.sync_copy(cbuf, cnt_hbm.at[r])


_compact = pl.kernel(
    _compact_body,
    out_type=[
        jax.ShapeDtypeStruct((ESLOT,), jnp.int32),
        jax.ShapeDtypeStruct((ESLOT,), jnp.int32),
        jax.ShapeDtypeStruct((ESLOT,), jnp.float32),
        jax.ShapeDtypeStruct((ESLOT,), jnp.float32),
        jax.ShapeDtypeStruct((NWORK, 16), jnp.int32),
    ],
    mesh=_MESH,
    compiler_params=pltpu.CompilerParams(needs_layout_passes=False),
    scratch_types=[
        pltpu.VMEM((N,), jnp.float32),
        pltpu.VMEM((N,), jnp.float32),
        pltpu.VMEM((CAPB,), jnp.int32),
        pltpu.VMEM((CAPB,), jnp.int32),
        pltpu.VMEM((CAPB,), jnp.float32),
        pltpu.VMEM((CAPB,), jnp.float32),
        pltpu.VMEM((16,), jnp.int32),
    ],
)


_GCH = 256


_WIDE = 2


def _gather_body(h_hbm, src_hbm, cnt_hbm, out_hbm,
                 idxall, r0, r1, cbuf,
                 g0, g1, o0, o1):
    c = lax.axis_index("c")
    s = lax.axis_index("s")
    r = c * NSUB + s
    rows = [r0, r1]
    gsem = [g0, g1]
    osem = [o0, o1]

    pltpu.sync_copy(src_hbm.at[pl.ds(r * CAP, CAP)], idxall)
    pltpu.sync_copy(cnt_hbm.at[r], cbuf)
    cnt = cbuf[pl.ds(0, 16)][0]
    nit = (cnt + _WIDE * _GCH - 1) // (_WIDE * _GCH)

    def step(i, _):
        hs = []
        for k in range(_WIDE):
            off = (i * _WIDE + k) * _GCH
            hs.append(pltpu.async_copy(
                h_hbm.at[idxall.at[pl.ds(off, _GCH)]], rows[k], gsem[k]))
        ws = []
        for k in range(_WIDE):
            off = (i * _WIDE + k) * _GCH
            hs[k].wait()
            ws.append(pltpu.async_copy(
                rows[k], out_hbm.at[pl.ds(r * CAP + off, _GCH)], osem[k]))
        for w in ws:
            w.wait()
        return 0

    lax.fori_loop(0, nit, step, 0)


_gather = pl.kernel(
    _gather_body,
    out_type=jax.ShapeDtypeStruct((ESLOT, D), jnp.float32),
    mesh=_MESH,
    compiler_params=pltpu.CompilerParams(needs_layout_passes=False),
    scratch_types=[
        pltpu.VMEM((CAP,), jnp.int32),
        pltpu.VMEM((_GCH, D), jnp.float32),
        pltpu.VMEM((_GCH, D), jnp.float32),
        pltpu.VMEM((16,), jnp.int32),
    ] + [pltpu.SemaphoreType.DMA] * (2 * _WIDE),
)


_AGR = N + NSUB * 8


def _scatter_body(msg_hbm, dst_hbm, cnt_hbm, out_hbm, aggr_s, zbuf,
                  idxall, r0, r1, cbuf, g0, g1):
    c = lax.axis_index("c")
    s = lax.axis_index("s")
    r = c * NSUB + s
    rows = [r0, r1]
    gsem = [g0, g1]

    def zlane(k, _):
        kk = k // 8
        ll = (k % 8) * 16
        zbuf[kk, pl.ds(ll, 16)] = jnp.zeros((16,), jnp.float32)
        return 0

    lax.fori_loop(0, 8 * (D // 16), zlane, 0)
    stripe = _AGR // NSUB
    for t in range(stripe // 8):
        pltpu.sync_copy(zbuf, aggr_s.at[pl.ds(s * stripe + t * 8, 8)])

    pltpu.sync_copy(dst_hbm.at[pl.ds(r * CAP, CAP)], idxall)
    pltpu.sync_copy(cnt_hbm.at[r], cbuf)
    cnt = cbuf[pl.ds(0, 16)][0]
    nit = (cnt + _WIDE * _GCH - 1) // (_WIDE * _GCH)
    plsc.subcore_barrier()

    def step(i, _):
        hs = []
        for k in range(_WIDE):
            off = (i * _WIDE + k) * _GCH
            hs.append(pltpu.async_copy(
                msg_hbm.at[pl.ds(r * CAP + off, _GCH)], rows[k], gsem[k]))
        for k in range(_WIDE):
            off = (i * _WIDE + k) * _GCH
            hs[k].wait()
            pltpu.sync_copy(rows[k], aggr_s.at[idxall.at[pl.ds(off, _GCH)]],
                            add=True)
        return 0

    lax.fori_loop(0, nit, step, 0)
    plsc.subcore_barrier()

    pltpu.sync_copy(aggr_s.at[pl.ds(s * ROWS_PER_SUB, ROWS_PER_SUB)],
                    out_hbm.at[pl.ds(c * N + s * ROWS_PER_SUB, ROWS_PER_SUB)])


_scatter = pl.kernel(
    _scatter_body,
    out_type=jax.ShapeDtypeStruct((B * N, D), jnp.float32),
    mesh=_MESH,
    compiler_params=pltpu.CompilerParams(needs_layout_passes=False),
    scratch_types=[
        pltpu.VMEM_SHARED((_AGR, D), jnp.float32),
        pltpu.VMEM((8, D), jnp.float32),
        pltpu.VMEM((CAP,), jnp.int32),
        pltpu.VMEM((_GCH, D), jnp.float32),
        pltpu.VMEM((_GCH, D), jnp.float32),
        pltpu.VMEM((16,), jnp.int32),
    ] + [pltpu.SemaphoreType.DMA] * _WIDE,
)


def _bdot(a, b):
    return jnp.dot(a, b, preferred_element_type=jnp.float32)


def _b32(v):
    return v.astype(jnp.bfloat16).astype(jnp.float32)


def _mlp_tail(a0, w1, b1, w2, b2, g, bn):
    h = jnp.maximum(a0, 0.0)
    h = jnp.maximum(_bdot(h, w1) + b1, 0.0)
    h = _bdot(h, w2) + b2
    if g is not None:
        mu = jnp.mean(h, axis=-1, keepdims=True)
        var = jnp.mean((h - mu) ** 2, axis=-1, keepdims=True)
        h = (h - mu) / jnp.sqrt(var + 1e-5) * g + bn
    return h


def _enc_kernel(x_ref, w0, b0, w1, b1, w2, b2, g, bn, o_ref):
    a0 = _bdot(x_ref[...], w0[...]) + b0[...]
    o_ref[...] = _mlp_tail(a0, w1[...], b1[...], w2[...], b2[...], g[...], bn[...])


def _edge_kernel(hs_ref, dx_ref, dy_ref, w0, b0, w1, b1, w2, b2, g, bn,
                 o_ref):
    dx = dx_ref[...]
    dy = dy_ref[...]
    eu = jnp.sqrt(dx * dx + dy * dy + jnp.float32(1e-8))
    xin = jnp.concatenate([hs_ref[...], dx, dy, eu], axis=-1)
    a0 = _bdot(xin, w0[...]) + b0[...]
    o_ref[...] = _mlp_tail(a0, w1[...], b1[...], w2[...], b2[...], g[...], bn[...])


def _node_kernel(h_ref, a_ref, w0, b0, w1, b1, w2, b2, g, bn, o_ref):
    xin = jnp.concatenate([h_ref[...], a_ref[...]], axis=-1)
    a0 = _bdot(xin, w0[...]) + b0[...]
    o_ref[...] = _mlp_tail(a0, w1[...], b1[...], w2[...], b2[...], g[...], bn[...])


def _dec_kernel(h_ref, w0, b0, w1, b1, w2, b2, o_ref):
    a0 = _bdot(h_ref[...], w0[...]) + b0[...]
    o_ref[...] = _mlp_tail(a0, w1[...], b1[...], w2[...], b2[...], None, None)


def _full(shape):
    return pl.BlockSpec(shape, lambda i: (0, 0))


_ROWB = 512


def _call_enc(xf, p):
    grid = (B * N // _ROWB,)
    return pl.pallas_call(
        _enc_kernel,
        grid=grid,
        in_specs=[pl.BlockSpec((_ROWB, 16), lambda i: (i, 0)),
                  _full((16, D)), _full((1, D)), _full((D, D)), _full((1, D)),
                  _full((D, D)), _full((1, D)), _full((1, D)), _full((1, D))],
        out_specs=pl.BlockSpec((_ROWB, D), lambda i: (i, 0)),
        out_shape=jax.ShapeDtypeStruct((B * N, D), jnp.float32),
    )(xf, p["l0"]["w"], p["l0"]["b"][None, :], p["l1"]["w"], p["l1"]["b"][None, :],
      p["l2"]["w"], p["l2"]["b"][None, :], p["ln"]["g"][None, :], p["ln"]["b"][None, :])


_EBLK = 512


def _call_edge(hs, dxc, dyc, p):
    grid = (ESLOT // _EBLK,)
    return pl.pallas_call(
        _edge_kernel,
        grid=grid,
        in_specs=[pl.BlockSpec((_EBLK, D), lambda i: (i, 0)),
                  pl.BlockSpec((_EBLK, 1), lambda i: (i, 0)),
                  pl.BlockSpec((_EBLK, 1), lambda i: (i, 0)),
                  _full((D + 3, D)), _full((1, D)), _full((D, D)),
                  _full((1, D)), _full((D, D)), _full((1, D)), _full((1, D)),
                  _full((1, D))],
        out_specs=pl.BlockSpec((_EBLK, D), lambda i: (i, 0)),
        out_shape=jax.ShapeDtypeStruct((ESLOT, D), jnp.float32),
    )(hs, dxc, dyc, p["l0"]["w"], p["l0"]["b"][None, :], p["l1"]["w"],
      p["l1"]["b"][None, :], p["l2"]["w"], p["l2"]["b"][None, :],
      p["ln"]["g"][None, :], p["ln"]["b"][None, :])


def _call_node(h, ag, p):
    grid = (B * N // _ROWB,)
    return pl.pallas_call(
        _node_kernel,
        grid=grid,
        in_specs=[pl.BlockSpec((_ROWB, D), lambda i: (i, 0)),
                  pl.BlockSpec((_ROWB, D), lambda i: (i, 0)),
                  _full((2 * D, D)), _full((1, D)), _full((D, D)),
                  _full((1, D)), _full((D, D)), _full((1, D)), _full((1, D)),
                  _full((1, D))],
        out_specs=pl.BlockSpec((_ROWB, D), lambda i: (i, 0)),
        out_shape=jax.ShapeDtypeStruct((B * N, D), jnp.float32),
    )(h, ag, p["l0"]["w"], p["l0"]["b"][None, :], p["l1"]["w"],
      p["l1"]["b"][None, :], p["l2"]["w"], p["l2"]["b"][None, :],
      p["ln"]["g"][None, :], p["ln"]["b"][None, :])


def _call_dec(h, p):
    grid = (B * N // _ROWB,)
    return pl.pallas_call(
        _dec_kernel,
        grid=grid,
        in_specs=[pl.BlockSpec((_ROWB, D), lambda i: (i, 0)),
                  _full((D, D)), _full((1, D)), _full((D, D)), _full((1, D)),
                  _full((D, 2)), _full((1, 2))],
        out_specs=pl.BlockSpec((_ROWB, 2), lambda i: (i, 0)),
        out_shape=jax.ShapeDtypeStruct((B * N, 2), jnp.float32),
    )(h, p["l0"]["w"], p["l0"]["b"][None, :], p["l1"]["w"], p["l1"]["b"][None, :],
      p["l2"]["w"], p["l2"]["b"][None, :])


def kernel(x, positions, params):
    xf = x.reshape(B * N, -1)
    px = positions[..., 0].reshape(B * N)
    py = positions[..., 1].reshape(B * N)

    h = _call_enc(xf, params["node_encoder"])
    src, dst, dxs, dys, cnts = _compact(px, py)
    dxc = dxs[:, None]
    dyc = dys[:, None]

    for lp in params["layers"]:
        hs = _gather(h, src, cnts)
        msg = _call_edge(hs, dxc, dyc, lp["edge_mlp"])
        ag = _scatter(msg, dst, cnts)
        h = _call_node(h, ag, lp["node_mlp"])

    out = _call_dec(h, params["node_decoder"])
    return out.reshape(B, N, 2)


# speedup vs baseline: 1.0941x; 1.0941x over previous
import functools

import jax
import jax.numpy as jnp
from jax import lax
from jax.experimental import pallas as pl
from jax.experimental.pallas import tpu as pltpu
from jax.experimental.pallas import tpu_sc as plsc

B = 2
N = 2048
D = 128
NCORE = 2
NSUB = 16
NWORK = NCORE * NSUB
ROWS_PER_SUB = N // NSUB
CAP = 3072
CAPB = CAP + ROWS_PER_SUB * 16 + 16
ESLOT = NWORK * CAP
RAD2_T = 0.0024999999441206455

_MESH = plsc.VectorSubcoreMesh(
    core_axis_name="c", subcore_axis_name="s", num_cores=NCORE, num_subcores=NSUB
)


def _compact_body(px_hbm, py_hbm, src_hbm, dst_hbm, dx_hbm, dy_hbm, cnt_hbm,
                  pxv, pyv, sbuf, dbuf, xbuf, ybuf, cbuf):
    c = lax.axis_index("c")
    s = lax.axis_index("s")
    r = c * NSUB + s

    pltpu.sync_copy(px_hbm.at[pl.ds(c * N, N)], pxv)
    pltpu.sync_copy(py_hbm.at[pl.ds(c * N, N)], pyv)

    zf = jnp.zeros((16,), jnp.float32)
    zi = jnp.zeros((16,), jnp.int32)
    dumm = jnp.full((16,), N, jnp.int32)

    def prefill(k, _):
        o = k * 16
        sbuf[pl.ds(o, 16)] = zi
        dbuf[pl.ds(o, 16)] = dumm
        xbuf[pl.ds(o, 16)] = zf
        ybuf[pl.ds(o, 16)] = zf
        return 0

    lax.fori_loop(0, CAP // 16, prefill, 0)

    iota = lax.iota(jnp.int32, 16)

    def row_body(i, cnt):
        row = s * ROWS_PER_SUB + i
        rowv = jnp.full((16,), row, jnp.int32)
        base = (row // 16) * 16
        lane = jnp.full((16,), row % 16, jnp.int32)
        pix = jnp.take_along_axis(pxv[pl.ds(base, 16)], lane, axis=0)
        piy = jnp.take_along_axis(pyv[pl.ds(base, 16)], lane, axis=0)

        def chunk_body(ch, cnt):
            j0 = ch * 16
            pjx = pxv[pl.ds(j0, 16)]
            pjy = pyv[pl.ds(j0, 16)]
            dx = pjx - pix
            dy = pjy - piy
            m = dx * dx + dy * dy + jnp.float32(1e-8)
            msk = m < jnp.float32(RAD2_T)
            plsc.store_compressed(sbuf.at[pl.ds(cnt, 16)],
                                  iota + (j0 + c * N), mask=msk)
            plsc.store_compressed(dbuf.at[pl.ds(cnt, 16)], rowv, mask=msk)
            plsc.store_compressed(xbuf.at[pl.ds(cnt, 16)], dx, mask=msk)
            plsc.store_compressed(ybuf.at[pl.ds(cnt, 16)], dy, mask=msk)
            return cnt + jnp.sum(msk.astype(jnp.int32))

        cnt = lax.fori_loop(0, N // 16, chunk_body, cnt)
        return jnp.minimum(cnt, CAP)

    cnt = lax.fori_loop(0, ROWS_PER_SUB, row_body, jnp.int32(0))

    sbuf[pl.ds(cnt, 16)] = zi
    dbuf[pl.ds(cnt, 16)] = dumm
    xbuf[pl.ds(cnt, 16)] = zf
    ybuf[pl.ds(cnt, 16)] = zf

    o = r * CAP
    pltpu.sync_copy(sbuf.at[pl.ds(0, CAP)], src_hbm.at[pl.ds(o, CAP)])
    pltpu.sync_copy(dbuf.at[pl.ds(0, CAP)], dst_hbm.at[pl.ds(o, CAP)])
    pltpu.sync_copy(xbuf.at[pl.ds(0, CAP)], dx_hbm.at[pl.ds(o, CAP)])
    pltpu.sync_copy(ybuf.at[pl.ds(0, CAP)], dy_hbm.at[pl.ds(o, CAP)])
    cbuf[pl.ds(0, 16)] = jnp.full((16,), cnt, jnp.int32)
    pltpu.sync_copy(cbuf, cnt_hbm.at[r])


_compact = pl.kernel(
    _compact_body,
    out_type=[
        jax.ShapeDtypeStruct((ESLOT,), jnp.int32),
        jax.ShapeDtypeStruct((ESLOT,), jnp.int32),
        jax.ShapeDtypeStruct((ESLOT,), jnp.float32),
        jax.ShapeDtypeStruct((ESLOT,), jnp.float32),
        jax.ShapeDtypeStruct((NWORK, 16), jnp.int32),
    ],
    mesh=_MESH,
    compiler_params=pltpu.CompilerParams(needs_layout_passes=False),
    scratch_types=[
        pltpu.VMEM((N,), jnp.float32),
        pltpu.VMEM((N,), jnp.float32),
        pltpu.VMEM((CAPB,), jnp.int32),
        pltpu.VMEM((CAPB,), jnp.int32),
        pltpu.VMEM((CAPB,), jnp.float32),
        pltpu.VMEM((CAPB,), jnp.float32),
        pltpu.VMEM((16,), jnp.int32),
    ],
)


_GCH = 256


_WIDE = 2


def _gather_body(h_hbm, src_hbm, cnt_hbm, out_hbm,
                 idxall, r0, r1, cbuf,
                 g0, g1, o0, o1):
    c = lax.axis_index("c")
    s = lax.axis_index("s")
    r = c * NSUB + s
    rows = [r0, r1]
    gsem = [g0, g1]
    osem = [o0, o1]

    pltpu.sync_copy(src_hbm.at[pl.ds(r * CAP, CAP)], idxall)
    pltpu.sync_copy(cnt_hbm.at[r], cbuf)
    cnt = cbuf[pl.ds(0, 16)][0]
    nit = (cnt + _WIDE * _GCH - 1) // (_WIDE * _GCH)

    def step(i, _):
        hs = []
        for k in range(_WIDE):
            off = (i * _WIDE + k) * _GCH
            hs.append(pltpu.async_copy(
                h_hbm.at[idxall.at[pl.ds(off, _GCH)]], rows[k], gsem[k]))
        ws = []
        for k in range(_WIDE):
            off = (i * _WIDE + k) * _GCH
            hs[k].wait()
            ws.append(pltpu.async_copy(
                rows[k], out_hbm.at[pl.ds(r * CAP + off, _GCH)], osem[k]))
        for w in ws:
            w.wait()
        return 0

    lax.fori_loop(0, nit, step, 0)


_gather = pl.kernel(
    _gather_body,
    out_type=jax.ShapeDtypeStruct((ESLOT, D), jnp.float32),
    mesh=_MESH,
    compiler_params=pltpu.CompilerParams(needs_layout_passes=False),
    scratch_types=[
        pltpu.VMEM((CAP,), jnp.int32),
        pltpu.VMEM((_GCH, D), jnp.float32),
        pltpu.VMEM((_GCH, D), jnp.float32),
        pltpu.VMEM((16,), jnp.int32),
    ] + [pltpu.SemaphoreType.DMA] * (2 * _WIDE),
)


_AGR = N + NSUB * 8


def _scatter_body(msg_hbm, dst_hbm, cnt_hbm, out_hbm, aggr_s, zbuf,
                  idxall, r0, r1, cbuf, g0, g1):
    c = lax.axis_index("c")
    s = lax.axis_index("s")
    r = c * NSUB + s
    rows = [r0, r1]
    gsem = [g0, g1]

    def zlane(k, _):
        kk = k // 8
        ll = (k % 8) * 16
        zbuf[kk, pl.ds(ll, 16)] = jnp.zeros((16,), jnp.float32)
        return 0

    lax.fori_loop(0, 8 * (D // 16), zlane, 0)
    stripe = _AGR // NSUB
    for t in range(stripe // 8):
        pltpu.sync_copy(zbuf, aggr_s.at[pl.ds(s * stripe + t * 8, 8)])

    pltpu.sync_copy(dst_hbm.at[pl.ds(r * CAP, CAP)], idxall)
    pltpu.sync_copy(cnt_hbm.at[r], cbuf)
    cnt = cbuf[pl.ds(0, 16)][0]
    nit = (cnt + _WIDE * _GCH - 1) // (_WIDE * _GCH)
    plsc.subcore_barrier()

    def step(i, _):
        hs = []
        for k in range(_WIDE):
            off = (i * _WIDE + k) * _GCH
            hs.append(pltpu.async_copy(
                msg_hbm.at[pl.ds(r * CAP + off, _GCH)], rows[k], gsem[k]))
        for k in range(_WIDE):
            off = (i * _WIDE + k) * _GCH
            hs[k].wait()
            pltpu.sync_copy(rows[k], aggr_s.at[idxall.at[pl.ds(off, _GCH)]],
                            add=True)
        return 0

    lax.fori_loop(0, nit, step, 0)
    plsc.subcore_barrier()

    pltpu.sync_copy(aggr_s.at[pl.ds(s * ROWS_PER_SUB, ROWS_PER_SUB)],
                    out_hbm.at[pl.ds(c * N + s * ROWS_PER_SUB, ROWS_PER_SUB)])


_scatter = pl.kernel(
    _scatter_body,
    out_type=jax.ShapeDtypeStruct((B * N, D), jnp.float32),
    mesh=_MESH,
    compiler_params=pltpu.CompilerParams(needs_layout_passes=False),
    scratch_types=[
        pltpu.VMEM_SHARED((_AGR, D), jnp.float32),
        pltpu.VMEM((8, D), jnp.float32),
        pltpu.VMEM((CAP,), jnp.int32),
        pltpu.VMEM((_GCH, D), jnp.float32),
        pltpu.VMEM((_GCH, D), jnp.float32),
        pltpu.VMEM((16,), jnp.int32),
    ] + [pltpu.SemaphoreType.DMA] * _WIDE,
)


def _bdot(a, b):
    return jnp.dot(a, b, preferred_element_type=jnp.float32)


def _b32(v):
    return v.astype(jnp.bfloat16).astype(jnp.float32)


def _mlp_tail(a0, w1, b1, w2, b2, g, bn):
    h = jnp.maximum(a0, 0.0)
    h = jnp.maximum(_bdot(h, w1) + b1, 0.0)
    h = _bdot(h, w2) + b2
    if g is not None:
        mu = jnp.mean(h, axis=-1, keepdims=True)
        var = jnp.mean((h - mu) ** 2, axis=-1, keepdims=True)
        h = (h - mu) / jnp.sqrt(var + 1e-5) * g + bn
    return h


def _enc_kernel(x_ref, w0, b0, w1, b1, w2, b2, g, bn, o_ref):
    a0 = _bdot(x_ref[...], w0[...]) + b0[...]
    o_ref[...] = _mlp_tail(a0, w1[...], b1[...], w2[...], b2[...], g[...], bn[...])


def _edge_kernel(hs_ref, dx_ref, dy_ref, w0, b0, w1, b1, w2, b2, g, bn,
                 o_ref):
    dx = dx_ref[...]
    dy = dy_ref[...]
    eu = jnp.sqrt(dx * dx + dy * dy + jnp.float32(1e-8))
    xin = jnp.concatenate([hs_ref[...], dx, dy, eu], axis=-1)
    a0 = _bdot(xin, w0[...]) + b0[...]
    o_ref[...] = _mlp_tail(a0, w1[...], b1[...], w2[...], b2[...], g[...], bn[...])


def _node_kernel(h_ref, a_ref, w0, b0, w1, b1, w2, b2, g, bn, o_ref):
    xin = jnp.concatenate([h_ref[...], a_ref[...]], axis=-1)
    a0 = _bdot(xin, w0[...]) + b0[...]
    o_ref[...] = _mlp_tail(a0, w1[...], b1[...], w2[...], b2[...], g[...], bn[...])


def _dec_kernel(h_ref, w0, b0, w1, b1, w2, b2, o_ref):
    a0 = _bdot(h_ref[...], w0[...]) + b0[...]
    o_ref[...] = _mlp_tail(a0, w1[...], b1[...], w2[...], b2[...], None, None)


def _full(shape):
    return pl.BlockSpec(shape, lambda i: (0, 0))


_ROWB = 512


def _call_enc(xf, p):
    grid = (B * N // _ROWB,)
    return pl.pallas_call(
        _enc_kernel,
        grid=grid,
        in_specs=[pl.BlockSpec((_ROWB, 16), lambda i: (i, 0)),
                  _full((16, D)), _full((1, D)), _full((D, D)), _full((1, D)),
                  _full((D, D)), _full((1, D)), _full((1, D)), _full((1, D))],
        out_specs=pl.BlockSpec((_ROWB, D), lambda i: (i, 0)),
        out_shape=jax.ShapeDtypeStruct((B * N, D), jnp.float32),
    )(xf, p["l0"]["w"], p["l0"]["b"][None, :], p["l1"]["w"], p["l1"]["b"][None, :],
      p["l2"]["w"], p["l2"]["b"][None, :], p["ln"]["g"][None, :], p["ln"]["b"][None, :])


_EBLK = 512


def _call_edge(hs, dxc, dyc, p):
    grid = (ESLOT // _EBLK,)
    return pl.pallas_call(
        _edge_kernel,
        grid=grid,
        in_specs=[pl.BlockSpec((_EBLK, D), lambda i: (i, 0)),
                  pl.BlockSpec((_EBLK, 1), lambda i: (i, 0)),
                  pl.BlockSpec((_EBLK, 1), lambda i: (i, 0)),
                  _full((D + 3, D)), _full((1, D)), _full((D, D)),
                  _full((1, D)), _full((D, D)), _full((1, D)), _full((1, D)),
                  _full((1, D))],
        out_specs=pl.BlockSpec((_EBLK, D), lambda i: (i, 0)),
        out_shape=jax.ShapeDtypeStruct((ESLOT, D), jnp.float32),
    )(hs, dxc, dyc, p["l0"]["w"], p["l0"]["b"][None, :], p["l1"]["w"],
      p["l1"]["b"][None, :], p["l2"]["w"], p["l2"]["b"][None, :],
      p["ln"]["g"][None, :], p["ln"]["b"][None, :])


def _call_node(h, ag, p):
    grid = (B * N // _ROWB,)
    return pl.pallas_call(
        _node_kernel,
        grid=grid,
        in_specs=[pl.BlockSpec((_ROWB, D), lambda i: (i, 0)),
                  pl.BlockSpec((_ROWB, D), lambda i: (i, 0)),
                  _full((2 * D, D)), _full((1, D)), _full((D, D)),
                  _full((1, D)), _full((D, D)), _full((1, D)), _full((1, D)),
                  _full((1, D))],
        out_specs=pl.BlockSpec((_ROWB, D), lambda i: (i, 0)),
        out_shape=jax.ShapeDtypeStruct((B * N, D), jnp.float32),
    )(h, ag, p["l0"]["w"], p["l0"]["b"][None, :], p["l1"]["w"],
      p["l1"]["b"][None, :], p["l2"]["w"], p["l2"]["b"][None, :],
      p["ln"]["g"][None, :], p["ln"]["b"][None, :])


def _call_dec(h, p):
    grid = (B * N // _ROWB,)
    return pl.pallas_call(
        _dec_kernel,
        grid=grid,
        in_specs=[pl.BlockSpec((_ROWB, D), lambda i: (i, 0)),
                  _full((D, D)), _full((1, D)), _full((D, D)), _full((1, D)),
                  _full((D, 2)), _full((1, 2))],
        out_specs=pl.BlockSpec((_ROWB, 2), lambda i: (i, 0)),
        out_shape=jax.ShapeDtypeStruct((B * N, 2), jnp.float32),
    )(h, p["l0"]["w"], p["l0"]["b"][None, :], p["l1"]["w"], p["l1"]["b"][None, :],
      p["l2"]["w"], p["l2"]["b"][None, :])


def kernel(x, positions, params):
    xf = x.reshape(B * N, -1)
    px = positions[..., 0].reshape(B * N)
    py = positions[..., 1].reshape(B * N)

    h = _call_enc(xf, params["node_encoder"])
    src, dst, dxs, dys, cnts = _compact(px, py)
    dxc = dxs[:, None]
    dyc = dys[:, None]

    for lp in params["layers"]:
        hs = _gather(h, src, cnts)
        msg = _call_edge(hs, dxc, dyc, lp["edge_mlp"])
        ag = _scatter(msg, dst, cnts)
        h = _call_node(h, ag, lp["node_mlp"])

    out = _call_dec(h, params["node_decoder"])
    return out.reshape(B, N, 2)
